# baseline (speedup 1.0000x reference)
import functools

import jax
import jax.numpy as jnp
from jax import lax
from jax.experimental import pallas as pl
from jax.experimental.pallas import tpu as pltpu
from jax.experimental.pallas import tpu_sc as plsc

N, M, U, D = 32, 4096, 2048, 256
CH = 32
NCH = M // CH
UCH = U // CH
RT = 8
RS = 6
G = 4
NJ = NCH // 2


@functools.lru_cache(maxsize=1)
def _build_sc_kernel():
    info = plsc.get_sparse_core_info()
    nc = info.num_cores
    ns = info.num_subcores
    mesh = plsc.VectorSubcoreMesh(core_axis_name="c", subcore_axis_name="s")

    @functools.partial(
        pl.kernel,
        out_type=jax.ShapeDtypeStruct((N * M, D), jnp.float32),
        mesh=mesh,
        scratch_types=[
            pltpu.VMEM((8, 128), jnp.int32),
            pltpu.VMEM((RT, CH, D), jnp.float32),
            pltpu.VMEM_SHARED((ns, RS, CH, D), jnp.float32),
            [pltpu.VMEM((CH,), jnp.int32)] * RT,
            [pltpu.VMEM((CH,), jnp.int32)] * RT,
            [pltpu.SemaphoreType.DMA] * RT,
            [pltpu.SemaphoreType.DMA] * RT,
            [pltpu.SemaphoreType.DMA] * RS,
            [pltpu.SemaphoreType.DMA] * RS,
        ],
    )
    def k(
        var_hbm, upd_hbm, idx_hbm, out_hbm,
        idx_v, buf, shbuf, sidx, didx, sgt, sst, sgs, sss,
    ):
        cid = lax.axis_index("c")
        sid = lax.axis_index("s")
        wid = sid * nc + cid
        pltpu.sync_copy(idx_hbm.at[wid], idx_v)
        v = idx_v[0, pl.ds(0, 16)]
        off = v[0]
        ln = v[1]
        end = off + ln
        vbase = wid * M
        ubase = wid * U

        def al(x):
            return pl.multiple_of(x, 8)

        def copy_cond(m):
            b = m * CH
            return jnp.logical_not(jnp.logical_and(off <= b, b + CH <= end))

        def tile_slot(j):
            return buf.at[j % RT]

        def sp_slot(j):
            return shbuf.at[sid, j % RS]

        paths = (
            (lambda j: 2 * j, tile_slot, sgt, sst, RT),
            (lambda j: 2 * j + 1, sp_slot, sgs, sss, RS),
        )

        def g_var(path, j):
            chunk_of, slot, sg, _, rp = path
            m = chunk_of(j)
            return pltpu.make_async_copy(
                var_hbm.at[pl.ds(al(vbase + m * CH), CH)], slot(j), sg[j % rp]
            )

        def s_out(path, j):
            chunk_of, slot, _, ss, rp = path
            m = chunk_of(j)
            return pltpu.make_async_copy(
                slot(j), out_hbm.at[pl.ds(al(vbase + m * CH), CH)], ss[j % rp]
            )

        for jj in range(NJ + G):
            for path in paths:
                chunk_of, _, _, _, rp = path
                if jj < NJ:
                    if jj >= rp:

                        @pl.when(copy_cond(chunk_of(jj - rp)))
                        def _(path=path, j=jj - rp):
                            s_out(path, j).wait()

                    @pl.when(copy_cond(chunk_of(jj)))
                    def _(path=path, j=jj):
                        g_var(path, j).start()

                if jj >= G:

                    @pl.when(copy_cond(chunk_of(jj - G)))
                    def _(path=path, j=jj - G):
                        g_var(path, j).wait()
                        s_out(path, j).start()

        for j in range(NJ - max(RT, RS), NJ):
            for path in paths:
                chunk_of, _, _, _, rp = path
                if j >= NJ - rp:

                    @pl.when(copy_cond(chunk_of(j)))
                    def _(path=path, j=j):
                        s_out(path, j).wait()

        nch = (ln + CH - 1) // CH
        iota = lax.iota(jnp.int32, 16)

        def g_upd(t):
            p = t % RT
            return pltpu.make_async_copy(upd_hbm.at[sidx[p]], buf.at[p], sgt[p])

        def s_upd(t):
            p = t % RT
            return pltpu.make_async_copy(buf.at[p], out_hbm.at[didx[p]], sst[p])

        for t in range(UCH + G):
            if t < UCH:
                if t >= RT:

                    @pl.when(t - RT < nch)
                    def _(t=t):
                        s_upd(t - RT).wait()

                @pl.when(t < nch)
                def _(t=t):
                    p = t % RT
                    for b in range(CH // 16):
                        q = jnp.minimum(t * CH + b * 16 + iota, ln - 1)
                        sidx[p][pl.ds(b * 16, 16)] = ubase + q
                        didx[p][pl.ds(b * 16, 16)] = vbase + off + q
                    g_upd(t).start()

            if t >= G:
                j = t - G

                @pl.when(j < nch)
                def _(j=j):
                    g_upd(j).wait()
                    s_upd(j).start()

        for j in range(UCH - RT, UCH):

            @pl.when(j < nch)
            def _(j=j):
                s_upd(j).wait()

    return k


def kernel(varRef, indice, updates, mask, reduce, axis):
    idx = indice.astype(jnp.int32)
    off = jnp.clip(idx[:, 0], 0, M)
    ln = jnp.clip(idx[:, 1], 0, M - off)
    ln = jnp.where(mask, ln, 0)
    idx3 = jnp.zeros((N, 8, 128), jnp.int32)
    idx3 = idx3.at[:, 0, 0].set(off).at[:, 0, 1].set(ln)
    out = _build_sc_kernel()(
        varRef.reshape(N * M, D), updates.reshape(N * U, D), idx3
    )
    return out.reshape(N, M, D)


# speedup vs baseline: 1.0041x; 1.0041x over previous
import functools

import jax
import jax.numpy as jnp
from jax import lax
from jax.experimental import pallas as pl
from jax.experimental.pallas import tpu as pltpu
from jax.experimental.pallas import tpu_sc as plsc

N, M, U, D = 32, 4096, 2048, 256
CH = 32
NCH = M // CH
UCH = U // CH
R1 = 8
G1 = 4
R2 = 6
G2 = 3


@functools.lru_cache(maxsize=1)
def _build_sc_kernel():
    info = plsc.get_sparse_core_info()
    nc = info.num_cores
    mesh = plsc.VectorSubcoreMesh(core_axis_name="c", subcore_axis_name="s")

    @functools.partial(
        pl.kernel,
        out_type=jax.ShapeDtypeStruct((N * M, D), jnp.float32),
        mesh=mesh,
        scratch_types=[
            pltpu.VMEM((8, 128), jnp.int32),
            pltpu.VMEM((R1, CH, D), jnp.float32),
            pltpu.VMEM((R2, CH, D), jnp.float32),
            [pltpu.VMEM((CH,), jnp.int32)] * R2,
            pltpu.VMEM((CH,), jnp.int32),
            pltpu.VMEM((CH,), jnp.int32),
            [pltpu.SemaphoreType.DMA] * R1,
            [pltpu.SemaphoreType.DMA] * R1,
            [pltpu.SemaphoreType.DMA] * R2,
            [pltpu.SemaphoreType.DMA] * R2,
        ],
    )
    def k(
        var_hbm, upd_hbm, idx_hbm, out_hbm,
        idx_v, buf1, buf2, didx, esidx, edidx, sg1, ss1, sg2, ss2,
    ):
        wid = lax.axis_index("s") * nc + lax.axis_index("c")
        pltpu.sync_copy(idx_hbm.at[wid], idx_v)
        v = idx_v[0, pl.ds(0, 16)]
        off = v[0]
        ln = v[1]
        end = off + ln
        vbase = wid * M
        ubase = wid * U
        nch = (ln + CH - 1) // CH
        iota = lax.iota(jnp.int32, 16)

        def al(x):
            return pl.multiple_of(x, 8)

        def copy_cond(m):
            b = m * CH
            return jnp.logical_not(jnp.logical_and(off <= b, b + CH <= end))

        def g_var(m):
            p = m % R1
            return pltpu.make_async_copy(
                var_hbm.at[pl.ds(al(vbase + m * CH), CH)], buf1.at[p], sg1[p]
            )

        def s_var(m):
            p = m % R1
            return pltpu.make_async_copy(
                buf1.at[p], out_hbm.at[pl.ds(al(vbase + m * CH), CH)], ss1[p]
            )

        k1_start = (end // CH) * CH

        def b_active(t):
            return jnp.logical_and(
                jnp.logical_and(t > 0, t < nch - 1),
                off + t * CH + CH <= k1_start,
            )

        def g_upd(t):
            p = t % R2
            return pltpu.make_async_copy(
                upd_hbm.at[pl.ds(al(ubase + t * CH), CH)], buf2.at[p], sg2[p]
            )

        def s_upd(t):
            p = t % R2
            return pltpu.make_async_copy(buf2.at[p], out_hbm.at[didx[p]], ss2[p])

        for mm in range(NCH + G1):
            if mm < NCH:
                if mm >= R1:

                    @pl.when(copy_cond(mm - R1))
                    def _(m=mm - R1):
                        s_var(m).wait()

                @pl.when(copy_cond(mm))
                def _(m=mm):
                    g_var(m).start()

            if mm >= G1:

                @pl.when(copy_cond(mm - G1))
                def _(m=mm - G1):
                    g_var(m).wait()
                    s_var(m).start()

            if mm % 2 == 0:
                t = mm // 2
                if t < UCH:
                    if t >= R2:

                        @pl.when(b_active(t - R2))
                        def _(t=t - R2):
                            s_upd(t).wait()

                    @pl.when(b_active(t))
                    def _(t=t):
                        p = t % R2
                        for b in range(CH // 16):
                            q = t * CH + b * 16 + iota
                            didx[p][pl.ds(b * 16, 16)] = vbase + off + q
                        g_upd(t).start()

                if t >= G2 and t - G2 < UCH:

                    @pl.when(b_active(t - G2))
                    def _(t=t - G2):
                        g_upd(t).wait()
                        s_upd(t).start()

        for m in range(NCH - R1, NCH):

            @pl.when(copy_cond(m))
            def _(m=m):
                s_var(m).wait()

        for t in range(UCH - R2, UCH):

            @pl.when(b_active(t))
            def _(t=t):
                s_upd(t).wait()

        def edge_chunk(t, slot):
            for b in range(CH // 16):
                q = jnp.minimum(t * CH + b * 16 + iota, ln - 1)
                esidx[pl.ds(b * 16, 16)] = ubase + q
                edidx[pl.ds(b * 16, 16)] = vbase + off + q
            g = pltpu.make_async_copy(upd_hbm.at[esidx], buf2.at[slot], sg2[slot])
            g.start()
            g.wait()
            s = pltpu.make_async_copy(buf2.at[slot], out_hbm.at[edidx], ss2[slot])
            s.start()
            s.wait()

        @pl.when(nch >= 1)
        def _():
            edge_chunk(0, 0)

        @pl.when(nch >= 2)
        def _():
            edge_chunk(nch - 1, 1)

        @pl.when(
            jnp.logical_and(nch >= 3, jnp.logical_not(b_active(nch - 2)))
        )
        def _():
            edge_chunk(nch - 2, 2)

    return k


def kernel(varRef, indice, updates, mask, reduce, axis):
    idx = indice.astype(jnp.int32)
    off = jnp.clip(idx[:, 0], 0, M)
    ln = jnp.clip(idx[:, 1], 0, M - off)
    ln = jnp.where(mask, ln, 0)
    idx3 = jnp.zeros((N, 8, 128), jnp.int32)
    idx3 = idx3.at[:, 0, 0].set(off).at[:, 0, 1].set(ln)
    out = _build_sc_kernel()(
        varRef.reshape(N * M, D), updates.reshape(N * U, D), idx3
    )
    return out.reshape(N, M, D)


# speedup vs baseline: 1.0098x; 1.0057x over previous
import functools

import jax
import jax.numpy as jnp
from jax import lax
from jax.experimental import pallas as pl
from jax.experimental.pallas import tpu as pltpu
from jax.experimental.pallas import tpu_sc as plsc

N, M, U, D = 32, 4096, 2048, 256
CH = 32
NCH = M // CH
UCH = U // CH
R = 14
G = 7


@functools.lru_cache(maxsize=1)
def _build_sc_kernel():
    info = plsc.get_sparse_core_info()
    nc = info.num_cores
    mesh = plsc.VectorSubcoreMesh(core_axis_name="c", subcore_axis_name="s")

    @functools.partial(
        pl.kernel,
        out_type=jax.ShapeDtypeStruct((N * M, D), jnp.float32),
        mesh=mesh,
        scratch_types=[
            pltpu.VMEM((8, 128), jnp.int32),
            pltpu.VMEM((R, CH, D), jnp.float32),
            [pltpu.VMEM((CH,), jnp.int32)] * R,
            [pltpu.VMEM((CH,), jnp.int32)] * R,
            [pltpu.SemaphoreType.DMA] * R,
            [pltpu.SemaphoreType.DMA] * R,
        ],
    )
    def k(var_hbm, upd_hbm, idx_hbm, out_hbm, idx_v, buf, sidx, didx, sg, ss):
        wid = lax.axis_index("s") * nc + lax.axis_index("c")
        pltpu.sync_copy(idx_hbm.at[wid], idx_v)
        v = idx_v[0, pl.ds(0, 16)]
        off = v[0]
        ln = v[1]
        end = off + ln
        vbase = wid * M
        ubase = wid * U

        def al(x):
            return pl.multiple_of(x, 8)

        def copy_cond(kk):
            b = kk * CH
            return jnp.logical_not(jnp.logical_and(off <= b, b + CH <= end))

        def g_var(kk):
            p = kk % R
            return pltpu.make_async_copy(
                var_hbm.at[pl.ds(al(vbase + kk * CH), CH)], buf.at[p], sg[p]
            )

        def s_out(kk):
            p = kk % R
            return pltpu.make_async_copy(
                buf.at[p], out_hbm.at[pl.ds(al(vbase + kk * CH), CH)], ss[p]
            )

        for kk in range(NCH + G):
            if kk < NCH:
                if kk >= R:

                    @pl.when(copy_cond(kk - R))
                    def _(j=kk - R):
                        s_out(j).wait()

                @pl.when(copy_cond(kk))
                def _(j=kk):
                    g_var(j).start()

            if kk >= G:

                @pl.when(copy_cond(kk - G))
                def _(j=kk - G):
                    g_var(j).wait()
                    s_out(j).start()

        for j in range(NCH - R, NCH):

            @pl.when(copy_cond(j))
            def _(j=j):
                s_out(j).wait()

        nch = (ln + CH - 1) // CH
        iota = lax.iota(jnp.int32, 16)

        def g_upd(t):
            p = t % R
            return pltpu.make_async_copy(upd_hbm.at[sidx[p]], buf.at[p], sg[p])

        def s_upd(t):
            p = t % R
            return pltpu.make_async_copy(buf.at[p], out_hbm.at[didx[p]], ss[p])

        for t in range(UCH + G):
            if t < UCH:
                if t >= R:

                    @pl.when(t - R < nch)
                    def _(j=t - R):
                        s_upd(j).wait()

                @pl.when(t < nch)
                def _(t=t):
                    p = t % R
                    for b in range(CH // 16):
                        q = jnp.minimum(t * CH + b * 16 + iota, ln - 1)
                        sidx[p][pl.ds(b * 16, 16)] = ubase + q
                        didx[p][pl.ds(b * 16, 16)] = vbase + off + q
                    g_upd(t).start()

            if t >= G:

                @pl.when(t - G < nch)
                def _(j=t - G):
                    g_upd(j).wait()
                    s_upd(j).start()

        for j in range(UCH - R, UCH):

            @pl.when(j < nch)
            def _(j=j):
                s_upd(j).wait()

    return k


def kernel(varRef, indice, updates, mask, reduce, axis):
    idx = indice.astype(jnp.int32)
    off = jnp.clip(idx[:, 0], 0, M)
    ln = jnp.clip(idx[:, 1], 0, M - off)
    ln = jnp.where(mask, ln, 0)
    idx3 = jnp.zeros((N, 8, 128), jnp.int32)
    idx3 = idx3.at[:, 0, 0].set(off).at[:, 0, 1].set(ln)
    out = _build_sc_kernel()(
        varRef.reshape(N * M, D), updates.reshape(N * U, D), idx3
    )
    return out.reshape(N, M, D)
